# Initial kernel scaffold; baseline (speedup 1.0000x reference)
import jax
import jax.numpy as jnp
from jax.experimental import pallas as pl


def kernel(x, edge_index, batch, bn_gamma, bn_beta, conv1_W, conv1_b, ln_gamma, ln_beta, lin_W, lin_b):
    raise NotImplementedError("write your pallas kernel here")


# speedup vs baseline: 23.9627x; 23.9627x over previous
import functools

import jax
import jax.numpy as jnp
from jax import lax
from jax.experimental import pallas as pl
from jax.experimental.pallas import tpu as pltpu
from jax.experimental.pallas import tpu_sc as plsc

N = 10000
E = 320000
F_IN = 128
H = 128
C = 16
G = 64
EPS = 1e-5

NC = 2
NS = 16
NW = NC * NS
K = 80
EPT = E // NW
CPT = EPT // K

_MESH = plsc.VectorSubcoreMesh(core_axis_name="c", subcore_axis_name="s",
                               num_cores=NC, num_subcores=NS)
_SC_PARAMS = pltpu.CompilerParams(use_tc_tiling_on_sc=False)


@functools.partial(
    pl.kernel,
    out_type=jax.ShapeDtypeStruct((2 * N,), jnp.float32),
    mesh=_MESH,
    compiler_params=_SC_PARAMS,
    scratch_types=[
        pltpu.VMEM_SHARED((N,), jnp.float32),
        pltpu.VMEM((CPT, K), jnp.int32),
        pltpu.VMEM((K,), jnp.float32),
    ],
)
def _deg_kernel(dst2d, zeros_n, out, deg_sh, didx_v, ones_v):
    c = lax.axis_index("c")
    s = lax.axis_index("s")
    w = c * NS + s

    @pl.when(s < 10)
    def _():
        pltpu.sync_copy(zeros_n.at[pl.ds(s * 1000, 1000)],
                        deg_sh.at[pl.ds(s * 1000, 1000)])

    for j in range(K // 16):
        ones_v[pl.ds(j * 16, 16)] = jnp.full((16,), 1.0, jnp.float32)

    pltpu.sync_copy(dst2d.at[pl.ds(w * CPT, CPT)], didx_v)
    plsc.subcore_barrier()

    def body(i, _):
        pltpu.sync_copy(ones_v, deg_sh.at[didx_v.at[i]], add=True)
        return ()

    lax.fori_loop(0, CPT, body, ())
    plsc.subcore_barrier()

    @pl.when(s < 10)
    def _():
        pltpu.sync_copy(deg_sh.at[pl.ds(s * 1000, 1000)],
                        out.at[pl.ds(c * N + s * 1000, 1000)])


@functools.partial(
    pl.kernel,
    out_type=jax.ShapeDtypeStruct((2 * N, H), jnp.float32),
    mesh=_MESH,
    compiler_params=_SC_PARAMS,
    scratch_types=[
        pltpu.VMEM_SHARED((N, H), jnp.float32),
        pltpu.VMEM((CPT, K), jnp.int32),
        pltpu.VMEM((CPT, K), jnp.int32),
        pltpu.VMEM((K, H), jnp.float32),
        pltpu.SemaphoreType.DMA,
    ],
)
def _scatter_kernel(h2, src2d, dst2d, zeros_nf, out, acc_sh, sidx_v, didx_v,
                    rows_v, gsem):
    c = lax.axis_index("c")
    s = lax.axis_index("s")
    w = c * NS + s

    @pl.when(s < 10)
    def _():
        pltpu.sync_copy(zeros_nf.at[pl.ds(s * 1000, 1000)],
                        acc_sh.at[pl.ds(s * 1000, 1000)])

    pltpu.sync_copy(src2d.at[pl.ds(w * CPT, CPT)], sidx_v)
    pltpu.sync_copy(dst2d.at[pl.ds(w * CPT, CPT)], didx_v)
    plsc.subcore_barrier()

    def body(i, _):
        pltpu.async_copy(h2.at[sidx_v.at[i]], rows_v, gsem).wait()
        pltpu.sync_copy(rows_v, acc_sh.at[didx_v.at[i]], add=True)
        return ()

    lax.fori_loop(0, CPT, body, ())
    plsc.subcore_barrier()

    @pl.when(s < 10)
    def _():
        pltpu.sync_copy(acc_sh.at[pl.ds(s * 1000, 1000)],
                        out.at[pl.ds(c * N + s * 1000, 1000)])


def _bnmm_body(x_ref, dega_ref, degb_ref, g_ref, b_ref, w_ref, h2_ref):
    x = x_ref[...]
    mean = jnp.mean(x, axis=0, keepdims=True)
    xc = x - mean
    var = jnp.mean(xc * xc, axis=0, keepdims=True)
    xh = xc * (g_ref[...] * lax.rsqrt(var + EPS)) + b_ref[...]
    deg = dega_ref[...] + degb_ref[...] + 1.0
    dinv = lax.rsqrt(deg)
    h = jnp.dot(xh, w_ref[...], preferred_element_type=jnp.float32)
    h2_ref[...] = h * dinv


def _final_body(accp_ref, h2_ref, dega_ref, degb_ref, cb_ref, lg_ref, lb_ref,
                batch_ref, lw_ref, lbias_ref, out_ref):
    acc = accp_ref[0:N, :] + accp_ref[N:2 * N, :] + h2_ref[...]
    deg = dega_ref[...] + degb_ref[...] + 1.0
    dinv = lax.rsqrt(deg)
    o = jnp.maximum(acc * dinv + cb_ref[...], 0.0)
    mu = jnp.mean(o, axis=-1, keepdims=True)
    oc = o - mu
    v = jnp.mean(oc * oc, axis=-1, keepdims=True)
    ln = oc * (lg_ref[...] * lax.rsqrt(v + EPS)) + lb_ref[...]
    ids = lax.broadcasted_iota(jnp.int32, (G, N), 0)
    oh = (ids == batch_ref[...]).astype(jnp.float32)
    sums = lax.dot_general(oh, ln, (((1,), (0,)), ((), ())),
                           preferred_element_type=jnp.float32)
    cnts = lax.dot_general(oh, jnp.ones((N, 1), jnp.float32),
                           (((1,), (0,)), ((), ())),
                           preferred_element_type=jnp.float32)
    pooled = sums / jnp.maximum(cnts, 1.0)
    out_ref[...] = jnp.dot(pooled, lw_ref[...],
                           preferred_element_type=jnp.float32) + lbias_ref[...]


def kernel(x, edge_index, batch, bn_gamma, bn_beta, conv1_W, conv1_b,
           ln_gamma, ln_beta, lin_W, lin_b):
    ei = edge_index.astype(jnp.int32)
    src2d = ei[0].reshape(E // K, K)
    dst2d = ei[1].reshape(E // K, K)
    zeros_n = jnp.zeros((N,), jnp.float32)
    zeros_nf = jnp.zeros((N, H), jnp.float32)

    degp = _deg_kernel(dst2d, zeros_n)
    dega = degp[:N].reshape(N, 1)
    degb = degp[N:].reshape(N, 1)

    h2 = pl.pallas_call(
        _bnmm_body,
        out_shape=jax.ShapeDtypeStruct((N, H), jnp.float32),
    )(x, dega, degb, bn_gamma, bn_beta, conv1_W)

    accp = _scatter_kernel(h2, src2d, dst2d, zeros_nf)

    logits = pl.pallas_call(
        _final_body,
        out_shape=jax.ShapeDtypeStruct((G, C), jnp.float32),
    )(accp, h2, dega, degb, conv1_b, ln_gamma, ln_beta,
      batch.astype(jnp.int32).reshape(1, N), lin_W, lin_b)
    return logits
